# baseline (speedup 1.0000x reference)
import functools

import jax
import jax.numpy as jnp
from jax import lax
from jax.experimental import pallas as pl
from jax.experimental.pallas import tpu as pltpu
from jax.experimental.pallas import tpu_sc as plsc

N_NODES = 10000
N_EDGES = 320000
D_FEAT = 128

NUM_CORES = 2
NUM_SUBCORES = 16
NW = NUM_CORES * NUM_SUBCORES
PER_W = N_EDGES // NW
CHUNK = 128
NBUF = 2
NFULL = PER_W // CHUNK
TAIL = PER_W - NFULL * CHUNK
NCHUNK = 80
NGROUP = NCHUNK // NBUF
PER_W_PAD = NCHUNK * CHUNK


def _gather_kernel(idx_hbm, px_hbm, out_hbm, idx_v, r0, r1, g0, g1, s0, s1):
    wid = lax.axis_index("s") * NUM_CORES + lax.axis_index("c")
    base = wid * PER_W
    rows = (r0, r1)
    gsems = (g0, g1)
    ssems = (s0, s1)

    pltpu.sync_copy(idx_hbm.at[wid], idx_v)

    def group(g, _):
        for b in range(NBUF):
            i = g * NBUF + b

            @pl.when(i >= NBUF)
            def _drain():
                @pl.when(i - NBUF < NFULL)
                def _dfull():
                    pltpu.make_async_copy(
                        rows[b], out_hbm.at[pl.ds(base, CHUNK)], ssems[b]
                    ).wait()

                @pl.when(i - NBUF == NFULL)
                def _dtail():
                    pltpu.make_async_copy(
                        rows[b].at[pl.ds(0, TAIL)],
                        out_hbm.at[pl.ds(base, TAIL)], ssems[b]
                    ).wait()

            pltpu.async_copy(px_hbm.at[idx_v.at[i]], rows[b], gsems[b]).wait()

            @pl.when(i < NFULL)
            def _full():
                pltpu.async_copy(rows[b],
                                 out_hbm.at[pl.ds(base + i * CHUNK, CHUNK)],
                                 ssems[b])

            @pl.when(i == NFULL)
            def _tail():
                pltpu.async_copy(rows[b].at[pl.ds(0, TAIL)],
                                 out_hbm.at[pl.ds(base + NFULL * CHUNK, TAIL)],
                                 ssems[b])
        return 0

    lax.fori_loop(0, NGROUP, group, 0)

    pltpu.make_async_copy(
        rows[NFULL % NBUF].at[pl.ds(0, TAIL)],
        out_hbm.at[pl.ds(base, TAIL)], ssems[NFULL % NBUF]).wait()


@jax.jit
def _pix_gather(ind_j, px):
    mesh = plsc.VectorSubcoreMesh(core_axis_name="c", subcore_axis_name="s")
    run = functools.partial(
        pl.kernel,
        mesh=mesh,
        out_type=jax.ShapeDtypeStruct((N_EDGES, D_FEAT), jnp.float32),
        scratch_types=[
            pltpu.VMEM((NCHUNK, CHUNK), jnp.int32),
            pltpu.VMEM((CHUNK, D_FEAT), jnp.float32),
            pltpu.VMEM((CHUNK, D_FEAT), jnp.float32),
            pltpu.SemaphoreType.DMA,
            pltpu.SemaphoreType.DMA,
            pltpu.SemaphoreType.DMA,
            pltpu.SemaphoreType.DMA,
        ],
    )(_gather_kernel)
    idx = ind_j.reshape(NW, PER_W)
    idx = jnp.pad(idx, ((0, 0), (0, PER_W_PAD - PER_W)))
    return run(idx.reshape(NW, NCHUNK, CHUNK), px)


def kernel(ind_2, px):
    return _pix_gather(ind_2[:, 1], px)


# speedup vs baseline: 3.2323x; 3.2323x over previous
import functools

import jax
import jax.numpy as jnp
from jax import lax
from jax.experimental import pallas as pl
from jax.experimental.pallas import tpu as pltpu
from jax.experimental.pallas import tpu_sc as plsc

N_NODES = 10000
N_EDGES = 320000
D_FEAT = 128

NUM_CORES = 2
NUM_SUBCORES = 16
NW = NUM_CORES * NUM_SUBCORES
PER_W = N_EDGES // NW
CHUNK = 128
NFULL = PER_W // CHUNK
TAIL = PER_W - NFULL * CHUNK
NCHUNK = NFULL + 1
PER_W_PAD = NCHUNK * CHUNK
FILL = N_NODES // 2


def _gather_kernel(idx_hbm, px_hbm, out_hbm, table_sp, idx_v, rows_v, sem):
    sid = lax.axis_index("s")
    wid = sid * NUM_CORES + lax.axis_index("c")
    base = wid * PER_W

    @pl.when(sid < 2)
    def _fill():
        pltpu.sync_copy(px_hbm.at[pl.ds(sid * FILL, FILL)],
                        table_sp.at[pl.ds(sid * FILL, FILL)])

    pltpu.sync_copy(idx_hbm.at[wid], idx_v)
    plsc.subcore_barrier()

    def body(i, _):
        pltpu.async_copy(table_sp.at[idx_v.at[i]], rows_v, sem).wait()
        @pl.when(i < NFULL)
        def _full():
            pltpu.sync_copy(rows_v, out_hbm.at[pl.ds(base + i * CHUNK, CHUNK)])

        @pl.when(i == NFULL)
        def _tail():
            pltpu.sync_copy(
                rows_v.at[pl.ds(0, TAIL)],
                out_hbm.at[pl.ds(base + NFULL * CHUNK, TAIL)],
            )
        return 0

    lax.fori_loop(0, NCHUNK, body, 0)


@jax.jit
def _pix_gather(ind_j, px):
    mesh = plsc.VectorSubcoreMesh(core_axis_name="c", subcore_axis_name="s")
    run = functools.partial(
        pl.kernel,
        mesh=mesh,
        out_type=jax.ShapeDtypeStruct((N_EDGES, D_FEAT), jnp.float32),
        scratch_types=[
            pltpu.VMEM_SHARED((N_NODES, D_FEAT), jnp.float32),
            pltpu.VMEM((NCHUNK, CHUNK), jnp.int32),
            pltpu.VMEM((CHUNK, D_FEAT), jnp.float32),
            pltpu.SemaphoreType.DMA,
        ],
    )(_gather_kernel)
    idx = ind_j.reshape(NW, PER_W)
    idx = jnp.pad(idx, ((0, 0), (0, PER_W_PAD - PER_W)))
    return run(idx.reshape(NW, NCHUNK, CHUNK), px)


def kernel(ind_2, px):
    return _pix_gather(ind_2[:, 1], px)


# speedup vs baseline: 4.5477x; 1.4069x over previous
import functools

import jax
import jax.numpy as jnp
from jax import lax
from jax.experimental import pallas as pl
from jax.experimental.pallas import tpu as pltpu
from jax.experimental.pallas import tpu_sc as plsc

N_NODES = 10000
N_EDGES = 320000
D_FEAT = 128

NUM_CORES = 2
NUM_SUBCORES = 16
NW = NUM_CORES * NUM_SUBCORES
PER_W = N_EDGES // NW
CHUNK = 128
NFULL = PER_W // CHUNK
TAIL = PER_W - NFULL * CHUNK
NCHUNK = 80
PER_W_PAD = NCHUNK * CHUNK
FILL = N_NODES // 2


def _gather_kernel(idx_hbm, px_hbm, out_hbm, table_sp, idx_v, rows_v,
                   rows_v2, sem, sem2):
    sid = lax.axis_index("s")
    wid = sid * NUM_CORES + lax.axis_index("c")
    base = wid * PER_W

    @pl.when(sid < 2)
    def _fill():
        pltpu.sync_copy(px_hbm.at[pl.ds(sid * FILL, FILL)],
                        table_sp.at[pl.ds(sid * FILL, FILL)])

    pltpu.sync_copy(idx_hbm.at[wid], idx_v)
    plsc.subcore_barrier()

    rows = (rows_v, rows_v2)
    sems = (sem, sem2)

    def start_gather(i, b):
        pltpu.async_copy(table_sp.at[idx_v.at[i]], rows[b], sems[b])

    def wait_gather(b):
        pltpu.make_async_copy(table_sp.at[idx_v.at[0]], rows[b],
                              sems[b]).wait()

    def scatter(i, b):
        @pl.when(i < NFULL)
        def _full():
            pltpu.sync_copy(rows[b],
                            out_hbm.at[pl.ds(base + i * CHUNK, CHUNK)])

        @pl.when(i == NFULL)
        def _tail():
            pltpu.sync_copy(rows[b].at[pl.ds(0, TAIL)],
                            out_hbm.at[pl.ds(base + NFULL * CHUNK, TAIL)])

    start_gather(0, 0)

    def body(p, _):
        i = 2 * p
        start_gather(i + 1, 1)
        wait_gather(0)
        scatter(i, 0)

        @pl.when(i + 2 < NCHUNK)
        def _next():
            start_gather(i + 2, 0)

        wait_gather(1)
        scatter(i + 1, 1)
        return 0

    lax.fori_loop(0, NCHUNK // 2, body, 0)


@jax.jit
def _pix_gather(ind_j, px):
    mesh = plsc.VectorSubcoreMesh(core_axis_name="c", subcore_axis_name="s")
    run = functools.partial(
        pl.kernel,
        mesh=mesh,
        out_type=jax.ShapeDtypeStruct((N_EDGES, D_FEAT), jnp.float32),
        scratch_types=[
            pltpu.VMEM_SHARED((N_NODES, D_FEAT), jnp.float32),
            pltpu.VMEM((NCHUNK, CHUNK), jnp.int32),
            pltpu.VMEM((CHUNK, D_FEAT), jnp.float32),
            pltpu.VMEM((CHUNK, D_FEAT), jnp.float32),
            pltpu.SemaphoreType.DMA,
            pltpu.SemaphoreType.DMA,
        ],
    )(_gather_kernel)
    idx = ind_j.reshape(NW, PER_W)
    idx = jnp.pad(idx, ((0, 0), (0, PER_W_PAD - PER_W)))
    return run(idx.reshape(NW, NCHUNK, CHUNK), px)


def kernel(ind_2, px):
    return _pix_gather(ind_2[:, 1], px)


# speedup vs baseline: 4.5923x; 1.0098x over previous
import functools

import jax
import jax.numpy as jnp
from jax import lax
from jax.experimental import pallas as pl
from jax.experimental.pallas import tpu as pltpu
from jax.experimental.pallas import tpu_sc as plsc

N_NODES = 10000
N_EDGES = 320000
D_FEAT = 128

NUM_CORES = 2
NUM_SUBCORES = 16
NW = NUM_CORES * NUM_SUBCORES
PER_W = N_EDGES // NW
CHUNK = 128
NFULL = PER_W // CHUNK
TAIL = PER_W - NFULL * CHUNK
NCHUNK = 80
PER_W_PAD = NCHUNK * CHUNK
FILL = N_NODES // 2


def _gather_kernel(idx_hbm, px_hbm, out_hbm, table_sp, idx_v, rows_v,
                   rows_v2, sem, sem2):
    sid = lax.axis_index("s")
    wid = sid * NUM_CORES + lax.axis_index("c")
    base = wid * PER_W

    @pl.when(sid < 2)
    def _fill():
        pltpu.sync_copy(px_hbm.at[pl.ds(sid * FILL, FILL)],
                        table_sp.at[pl.ds(sid * FILL, FILL)])

    pltpu.sync_copy(idx_hbm.at[pl.ds(base, PER_W)],
                    idx_v.at[pl.ds(0, PER_W)])
    zeros = jnp.zeros((16,), jnp.int32)
    for j in range((PER_W_PAD - PER_W) // 16):
        idx_v[pl.ds(PER_W + 16 * j, 16)] = zeros
    plsc.subcore_barrier()

    rows = (rows_v, rows_v2)
    sems = (sem, sem2)

    def start_gather(i, b):
        pltpu.async_copy(table_sp.at[idx_v.at[pl.ds(i * CHUNK, CHUNK)]],
                         rows[b], sems[b])

    def wait_gather(b):
        pltpu.make_async_copy(table_sp.at[idx_v.at[pl.ds(0, CHUNK)]],
                              rows[b], sems[b]).wait()

    def scatter(i, b):
        @pl.when(i < NFULL)
        def _full():
            pltpu.sync_copy(rows[b],
                            out_hbm.at[pl.ds(base + i * CHUNK, CHUNK)])

        @pl.when(i == NFULL)
        def _tail():
            pltpu.sync_copy(rows[b].at[pl.ds(0, TAIL)],
                            out_hbm.at[pl.ds(base + NFULL * CHUNK, TAIL)])

    start_gather(0, 0)

    def body(p, _):
        i = 2 * p
        start_gather(i + 1, 1)
        wait_gather(0)
        scatter(i, 0)

        @pl.when(i + 2 < NCHUNK)
        def _next():
            start_gather(i + 2, 0)

        wait_gather(1)
        scatter(i + 1, 1)
        return 0

    lax.fori_loop(0, NCHUNK // 2, body, 0)


@jax.jit
def _pix_gather(ind_j, px):
    mesh = plsc.VectorSubcoreMesh(core_axis_name="c", subcore_axis_name="s")
    run = functools.partial(
        pl.kernel,
        mesh=mesh,
        out_type=jax.ShapeDtypeStruct((N_EDGES, D_FEAT), jnp.float32),
        scratch_types=[
            pltpu.VMEM_SHARED((N_NODES, D_FEAT), jnp.float32),
            pltpu.VMEM((PER_W_PAD,), jnp.int32),
            pltpu.VMEM((CHUNK, D_FEAT), jnp.float32),
            pltpu.VMEM((CHUNK, D_FEAT), jnp.float32),
            pltpu.SemaphoreType.DMA,
            pltpu.SemaphoreType.DMA,
        ],
    )(_gather_kernel)
    return run(ind_j, px)


def kernel(ind_2, px):
    return _pix_gather(ind_2[:, 1], px)
